# Initial kernel scaffold; baseline (speedup 1.0000x reference)
import jax
import jax.numpy as jnp
from jax.experimental import pallas as pl


def kernel(eigen_vectors, eigen_values, edge_attr, edge_index, batch, params):
    raise NotImplementedError("write your pallas kernel here")


# speedup vs baseline: 1.6859x; 1.6859x over previous
import jax
import jax.numpy as jnp
from jax.experimental import pallas as pl

_N = 10000
_K = 8
_E = 160000
_C = 32
_NHEAD = 4
_DH = 8
_NG = 1250
_RB = 4000
_NB = 400


def _lin_stats_kern(din, zero_mask):
    def kern(x_ref, m_ref, W_ref, b_ref, y_ref, s_ref, q_ref):
        x = x_ref[...]
        W = W_ref[...]
        if din == 1:
            y = x * W + b_ref[...]
        else:
            y = jnp.dot(x, W, preferred_element_type=jnp.float32) + b_ref[...]
        m = m_ref[...]
        ym = y * m
        if zero_mask:
            y_ref[...] = ym
            s_ref[...] = jnp.sum(ym, axis=0, keepdims=True)[None]
            q_ref[...] = jnp.sum(ym * ym, axis=0, keepdims=True)[None]
        else:
            y_ref[...] = y
            s_ref[...] = jnp.sum(ym, axis=0, keepdims=True)[None]
            q_ref[...] = jnp.sum(ym * y, axis=0, keepdims=True)[None]
    return kern


def _gin_mlp_kern(din):
    def kern(x_ref, agg_ref, m_ref, eps_ref, W1_ref, b1_ref, W2_ref, b2_ref,
             y_ref, s_ref, q_ref):
        h = (1.0 + eps_ref[...]) * x_ref[...] + agg_ref[...]
        if din == 1:
            t = jax.nn.relu(h * W1_ref[...] + b1_ref[...])
        else:
            t = jax.nn.relu(jnp.dot(h, W1_ref[...],
                                    preferred_element_type=jnp.float32)
                            + b1_ref[...])
        y = jnp.dot(t, W2_ref[...], preferred_element_type=jnp.float32) + b2_ref[...]
        ym = y * m_ref[...]
        y_ref[...] = ym
        s_ref[...] = jnp.sum(ym, axis=0, keepdims=True)[None]
        q_ref[...] = jnp.sum(ym * ym, axis=0, keepdims=True)[None]
    return kern


def _bn_kern(relu, residual):
    def kern(y_ref, mean_ref, var_ref, g_ref, b_ref, *rest):
        if residual:
            prev_ref, o_ref = rest
        else:
            (o_ref,) = rest
        xn = ((y_ref[...] - mean_ref[0])
              / jnp.sqrt(var_ref[0] + 1e-5)) * g_ref[...] + b_ref[...]
        if relu:
            xn = jax.nn.relu(xn)
        if residual:
            xn = xn + prev_ref[...]
        o_ref[...] = xn
    return kern


def _edge_kern(ea_ref, W0_ref, b0_ref, W1_ref, b1_ref, W2_ref, b2_ref,
               e0_ref, e1_ref, e2_ref):
    ea = ea_ref[...]
    e0_ref[...] = jnp.dot(ea, W0_ref[...], preferred_element_type=jnp.float32) + b0_ref[...]
    e1_ref[...] = jnp.dot(ea, W1_ref[...], preferred_element_type=jnp.float32) + b1_ref[...]
    e2_ref[...] = jnp.dot(ea, W2_ref[...], preferred_element_type=jnp.float32) + b2_ref[...]


def _ln(x, g, b):
    mu = jnp.mean(x, axis=-1, keepdims=True)
    var = jnp.mean((x - mu) ** 2, axis=-1, keepdims=True)
    return (x - mu) / jnp.sqrt(var + 1e-5) * g + b


def _rho_kern(x_ref, pos_ref, m_ref, *refs):
    pr = refs[:33]
    h_ref, s_ref, q_ref = refs[33], refs[34], refs[35]
    x = x_ref[...] + pos_ref[...]
    B = x.shape[0]
    mask = m_ref[...]
    scale = 1.0 / jnp.sqrt(jnp.float32(_DH))
    for l in range(2):
        (Wq, bq, Wk, bk, Wv, bv, Wo, bo, F1, f1b, F2, f2b,
         ln1g, ln1b, ln2g, ln2b) = [r[...] for r in pr[l * 16:(l + 1) * 16]]
        xf = x.reshape(B * _K, _C)
        q = (jnp.dot(xf, Wq, preferred_element_type=jnp.float32) + bq).reshape(B, _K, _C)
        k = (jnp.dot(xf, Wk, preferred_element_type=jnp.float32) + bk).reshape(B, _K, _C)
        v = (jnp.dot(xf, Wv, preferred_element_type=jnp.float32) + bv).reshape(B, _K, _C)
        heads = []
        for hh in range(_NHEAD):
            qh = q[..., hh * _DH:(hh + 1) * _DH]
            kh = k[..., hh * _DH:(hh + 1) * _DH]
            vh = v[..., hh * _DH:(hh + 1) * _DH]
            rows = [jnp.sum(qh[:, qi:qi + 1, :] * kh, axis=-1) * scale
                    for qi in range(_K)]
            sc = jnp.stack(rows, axis=1)
            sc = jnp.where(mask[:, None, :] > 0.5, sc, -1e9)
            sc = sc - jnp.max(sc, axis=-1, keepdims=True)
            ex = jnp.exp(sc)
            a = ex / jnp.sum(ex, axis=-1, keepdims=True)
            outs = [jnp.sum(a[:, qi, :, None] * vh, axis=1)
                    for qi in range(_K)]
            heads.append(jnp.stack(outs, axis=1))
        o = jnp.concatenate(heads, axis=-1).reshape(B * _K, _C)
        o = (jnp.dot(o, Wo, preferred_element_type=jnp.float32) + bo).reshape(B, _K, _C)
        x = _ln(x + o, ln1g, ln1b)
        xf = x.reshape(B * _K, _C)
        f = jnp.dot(jax.nn.relu(jnp.dot(xf, F1, preferred_element_type=jnp.float32) + f1b),
                    F2, preferred_element_type=jnp.float32) + f2b
        x = _ln(x + f.reshape(B, _K, _C), ln2g, ln2b)
    pooled = jnp.sum(x, axis=1)
    h = jnp.dot(pooled, pr[32][...], preferred_element_type=jnp.float32)
    h_ref[...] = h
    s_ref[...] = jnp.sum(h, axis=0, keepdims=True)[None]
    q_ref[...] = jnp.sum(h * h, axis=0, keepdims=True)[None]


def _full(shape):
    return pl.BlockSpec(shape, lambda i: tuple(0 for _ in shape))


def _call_lin_stats(x, m, W, b, din, zero_mask, rb=_RB):
    R = x.shape[0]
    G = R // rb
    return pl.pallas_call(
        _lin_stats_kern(din, zero_mask),
        grid=(G,),
        in_specs=[
            pl.BlockSpec((rb, din), lambda i: (i, 0)),
            pl.BlockSpec((rb, 1), lambda i: (i, 0)),
            _full(W.shape),
            _full((1, _C)),
        ],
        out_specs=[
            pl.BlockSpec((rb, _C), lambda i: (i, 0)),
            pl.BlockSpec((1, 1, _C), lambda i: (i, 0, 0)),
            pl.BlockSpec((1, 1, _C), lambda i: (i, 0, 0)),
        ],
        out_shape=[
            jax.ShapeDtypeStruct((R, _C), jnp.float32),
            jax.ShapeDtypeStruct((G, 1, _C), jnp.float32),
            jax.ShapeDtypeStruct((G, 1, _C), jnp.float32),
        ],
    )(x, m, W, b.reshape(1, _C))


def _call_gin_mlp(x, agg, m, eps, W1, b1, W2, b2, din, rb=_RB):
    R = x.shape[0]
    G = R // rb
    return pl.pallas_call(
        _gin_mlp_kern(din),
        grid=(G,),
        in_specs=[
            pl.BlockSpec((rb, din), lambda i: (i, 0)),
            pl.BlockSpec((rb, din), lambda i: (i, 0)),
            pl.BlockSpec((rb, 1), lambda i: (i, 0)),
            _full((1, 1)),
            _full(W1.shape),
            _full((1, _C)),
            _full((_C, _C)),
            _full((1, _C)),
        ],
        out_specs=[
            pl.BlockSpec((rb, _C), lambda i: (i, 0)),
            pl.BlockSpec((1, 1, _C), lambda i: (i, 0, 0)),
            pl.BlockSpec((1, 1, _C), lambda i: (i, 0, 0)),
        ],
        out_shape=[
            jax.ShapeDtypeStruct((R, _C), jnp.float32),
            jax.ShapeDtypeStruct((G, 1, _C), jnp.float32),
            jax.ShapeDtypeStruct((G, 1, _C), jnp.float32),
        ],
    )(x, agg, m, eps.reshape(1, 1), W1, b1.reshape(1, _C), W2, b2.reshape(1, _C))


def _call_bn(y, mean, var, g, b, prev=None, relu=True, rb=_RB):
    R = y.shape[0]
    G = R // rb
    per = G // mean.shape[0]
    in_specs = [
        pl.BlockSpec((rb, _C), lambda i: (i, 0)),
        pl.BlockSpec((1, 1, _C), lambda i, per=per: (i // per, 0, 0)),
        pl.BlockSpec((1, 1, _C), lambda i, per=per: (i // per, 0, 0)),
        _full((1, _C)),
        _full((1, _C)),
    ]
    args = [y, mean, var, g.reshape(1, _C), b.reshape(1, _C)]
    if prev is not None:
        in_specs.append(pl.BlockSpec((rb, _C), lambda i: (i, 0)))
        args.append(prev)
    return pl.pallas_call(
        _bn_kern(relu, prev is not None),
        grid=(G,),
        in_specs=in_specs,
        out_specs=pl.BlockSpec((rb, _C), lambda i: (i, 0)),
        out_shape=jax.ShapeDtypeStruct((R, _C), jnp.float32),
    )(*args)


def kernel(eigen_vectors, eigen_values, edge_attr, edge_index, batch, params):
    n, k = eigen_vectors.shape
    size = jax.ops.segment_sum(jnp.ones((n,), jnp.int32), batch, num_segments=_NG)
    mask = (jnp.arange(k)[None, :] < size[:, None])[batch]
    maskf = mask.astype(jnp.float32)

    m_enc = maskf.reshape(-1, 1)
    cnt_enc = jnp.maximum(m_enc.sum(), 1.0)
    pe = params['enc']
    y, s, q = _call_lin_stats(eigen_values.reshape(-1, 1), m_enc,
                              pe['W1'], pe['b1'], din=1, zero_mask=False)
    mean = (s.sum(axis=0) / cnt_enc).reshape(1, 1, _C)
    var = (q.sum(axis=0) / cnt_enc).reshape(1, 1, _C) - mean * mean
    h = _call_bn(y, mean, var, pe['g1'], pe['be1'], relu=True)
    y, s, q = _call_lin_stats(h, m_enc, pe['W2'], pe['b2'], din=_C, zero_mask=False)
    mean = (s.sum(axis=0) / cnt_enc).reshape(1, 1, _C)
    var = (q.sum(axis=0) / cnt_enc).reshape(1, 1, _C) - mean * mean
    pos = _call_bn(y, mean, var, pe['g2'], pe['be2'], relu=True).reshape(n, k, _C)

    ph = params['phi']
    EB = _RB
    EG = _E // EB
    e0, e1, e2 = pl.pallas_call(
        _edge_kern,
        grid=(EG,),
        in_specs=[pl.BlockSpec((EB, 4), lambda i: (i, 0)),
                  _full((4, 1)), _full((1, 1)),
                  _full((4, _C)), _full((1, _C)),
                  _full((4, _C)), _full((1, _C))],
        out_specs=[pl.BlockSpec((EB, 1), lambda i: (i, 0)),
                   pl.BlockSpec((EB, _C), lambda i: (i, 0)),
                   pl.BlockSpec((EB, _C), lambda i: (i, 0))],
        out_shape=[jax.ShapeDtypeStruct((_E, 1), jnp.float32),
                   jax.ShapeDtypeStruct((_E, _C), jnp.float32),
                   jax.ShapeDtypeStruct((_E, _C), jnp.float32)],
    )(edge_attr, ph[0]['We'], ph[0]['be'].reshape(1, 1),
      ph[1]['We'], ph[1]['be'].reshape(1, _C),
      ph[2]['We'], ph[2]['be'].reshape(1, _C))
    es = [e0, e1, e2]

    src, dst = edge_index[0], edge_index[1]
    evT = eigen_vectors.T
    x16 = jnp.concatenate([evT, -evT], axis=0)[..., None]
    m8 = maskf.T
    m16 = jnp.concatenate([m8, m8], axis=0).reshape(-1, 1)
    cnt_half = jnp.maximum(m8.sum(), 1.0)
    prev = jnp.zeros((16 * n, _C), jnp.float32)
    for li, p in enumerate(ph):
        din = 1 if li == 0 else _C
        msg = jax.nn.relu(x16[:, src, :] + es[li][None, :, :])
        agg = jnp.zeros_like(x16).at[:, dst, :].add(msg)
        y, s, q = _call_gin_mlp(x16.reshape(16 * n, din), agg.reshape(16 * n, din),
                                m16, p['eps'], p['W1'], p['b1'], p['W2'], p['b2'],
                                din)
        G = (16 * n) // _RB
        sh = s.reshape(2, G // 2, _C).sum(axis=1).reshape(2, 1, _C)
        qh = q.reshape(2, G // 2, _C).sum(axis=1).reshape(2, 1, _C)
        meanh = sh / cnt_half
        varh = qh / cnt_half - meanh * meanh
        x_new = _call_bn(y, meanh, varh, p['bng'], p['bnb'], prev=prev, relu=True)
        prev = x_new
        x16 = x_new.reshape(16, n, _C)
    hsum = (x16[:8] + x16[8:]).transpose(1, 0, 2)

    flat = []
    for p in params['rho']:
        flat += [p['Wq'], p['bq'].reshape(1, _C), p['Wk'], p['bk'].reshape(1, _C),
                 p['Wv'], p['bv'].reshape(1, _C), p['Wo'], p['bo'].reshape(1, _C),
                 p['F1'], p['f1b'].reshape(1, _C), p['F2'], p['f2b'].reshape(1, _C),
                 p['ln1g'].reshape(1, _C), p['ln1b'].reshape(1, _C),
                 p['ln2g'].reshape(1, _C), p['ln2b'].reshape(1, _C)]
    flat.append(params['out']['W'])
    NG2 = n // _NB
    in_specs = [pl.BlockSpec((_NB, _K, _C), lambda i: (i, 0, 0)),
                pl.BlockSpec((_NB, _K, _C), lambda i: (i, 0, 0)),
                pl.BlockSpec((_NB, _K), lambda i: (i, 0))]
    in_specs += [_full(a.shape) for a in flat]
    hh, s, q = pl.pallas_call(
        _rho_kern,
        grid=(NG2,),
        in_specs=in_specs,
        out_specs=[pl.BlockSpec((_NB, _C), lambda i: (i, 0)),
                   pl.BlockSpec((1, 1, _C), lambda i: (i, 0, 0)),
                   pl.BlockSpec((1, 1, _C), lambda i: (i, 0, 0))],
        out_shape=[jax.ShapeDtypeStruct((n, _C), jnp.float32),
                   jax.ShapeDtypeStruct((NG2, 1, _C), jnp.float32),
                   jax.ShapeDtypeStruct((NG2, 1, _C), jnp.float32)],
    )(hsum, pos, maskf, *flat)
    mu = (s.sum(axis=0) / n).reshape(1, 1, _C)
    var = (q.sum(axis=0) / n).reshape(1, 1, _C) - mu * mu
    po = params['out']
    return _call_bn(hh, mu, var, po['g'], po['b'], relu=False, rb=2000)
